# baseline (speedup 1.0000x reference)
import functools

import jax
import jax.numpy as jnp
from jax import lax
from jax.experimental import pallas as pl
from jax.experimental.pallas import tpu as pltpu
from jax.experimental.pallas import tpu_sc as plsc

N_TOPICS = 100000
N_CLASSES = 128
BATCH = 16384

NC, NS, L = 2, 16, 16
NW = NC * NS
BPW = BATCH // NW
CH = 128
CHS = (32, 32, 128, 128, 128, 32, 32)
OFFS = (0, 32, 64, 192, 320, 448, 480)
NCH = len(CHS)
NB = 3
AHEAD = 2
CREG = N_CLASSES // L

_mesh = plsc.VectorSubcoreMesh(core_axis_name="c", subcore_axis_name="s")


@functools.partial(
    pl.kernel,
    out_type=jax.ShapeDtypeStruct((BATCH, N_CLASSES), jnp.float32),
    mesh=_mesh,
    scratch_types=[
        pltpu.VMEM((BPW,), jnp.int32),
        pltpu.VMEM((BPW,), jnp.float32),
        pltpu.VMEM((NB, CH, N_CLASSES), jnp.float32),
        pltpu.VMEM((NB, CH, N_CLASSES), jnp.float32),
        pltpu.SemaphoreType.DMA,
        [pltpu.SemaphoreType.DMA] * NB,
        [pltpu.SemaphoreType.DMA] * NB,
        [pltpu.SemaphoreType.DMA] * NB,
    ],
)
def _calibrate(logits_hbm, topics_hbm, loga_hbm, b_hbm, out_hbm,
               idx_all, loga_all, x_v, b_v, sem_l, sem_x, sem_b, sem_o):
    wid = lax.axis_index("s") * NC + lax.axis_index("c")
    base = wid * BPW

    pltpu.sync_copy(topics_hbm.at[pl.ds(base, BPW)], idx_all)
    cp_l = pltpu.async_copy(loga_hbm.at[idx_all], loga_all, sem_l)

    outs = [None] * NB

    def issue(ch):
        k = ch % NB
        if outs[k] is not None:
            outs[k].wait()
            outs[k] = None
        n = CHS[ch]
        off = base + OFFS[ch]
        cpx = pltpu.async_copy(logits_hbm.at[pl.ds(off, n)],
                               x_v.at[k].at[pl.ds(0, n)], sem_x[k])
        cpb = pltpu.async_copy(b_hbm.at[idx_all.at[pl.ds(OFFS[ch], n)]],
                               b_v.at[k].at[pl.ds(0, n)], sem_b[k])
        return cpx, cpb

    pending = {}
    for ch in range(min(AHEAD, NCH)):
        pending[ch] = issue(ch)
    cp_l.wait()
    for ch in range(NCH):
        if ch + AHEAD < NCH:
            pending[ch + AHEAD] = issue(ch + AHEAD)
        k = ch % NB
        cpx, cpb = pending.pop(ch)
        cpx.wait()
        cpb.wait()

        def pair_body(i, carry):
            r0 = i * 2
            g = (r0 // L) * L
            sv = jnp.exp(loga_all[pl.ds(OFFS[ch] + g, L)])
            for j in range(2):
                r = r0 + j
                iv = jnp.broadcast_to(r - g, (L,))
                s = lax.gather(
                    sv, iv[:, None],
                    dimension_numbers=lax.GatherDimensionNumbers(
                        offset_dims=(), collapsed_slice_dims=(0,),
                        start_index_map=(0,)),
                    slice_sizes=(1,),
                    mode=lax.GatherScatterMode.PROMISE_IN_BOUNDS)
                for c in range(CREG):
                    sl = (k, r, pl.ds(c * L, L))
                    plsc.addupdate(b_v.at[sl], x_v[sl] * s)
            return carry

        lax.fori_loop(0, CHS[ch] // 2, pair_body, 0)
        outs[k] = pltpu.async_copy(b_v.at[k].at[pl.ds(0, CHS[ch])],
                                   out_hbm.at[pl.ds(base + OFFS[ch], CHS[ch])],
                                   sem_o[k])
    for cp in outs:
        if cp is not None:
            cp.wait()


def kernel(logits, topics, loga, b):
    if topics.dtype != jnp.int32:
        topics = topics.astype(jnp.int32)
    return _calibrate(logits, topics, loga, b)


# speedup vs baseline: 1.0212x; 1.0212x over previous
import functools

import jax
import jax.numpy as jnp
from jax import lax
from jax.experimental import pallas as pl
from jax.experimental.pallas import tpu as pltpu
from jax.experimental.pallas import tpu_sc as plsc

N_TOPICS = 100000
N_CLASSES = 128
BATCH = 16384

NC, NS, L = 2, 16, 16
NW = NC * NS
BPW = BATCH // NW
CH = 128
CHS = (32, 32, 128, 128, 128, 64)
OFFS = (0, 32, 64, 192, 320, 448)
NCH = len(CHS)
NB = 3
AHEAD = 2
CREG = N_CLASSES // L

_mesh = plsc.VectorSubcoreMesh(core_axis_name="c", subcore_axis_name="s")


@functools.partial(
    pl.kernel,
    out_type=jax.ShapeDtypeStruct((BATCH, N_CLASSES), jnp.float32),
    mesh=_mesh,
    scratch_types=[
        pltpu.VMEM((BPW,), jnp.int32),
        pltpu.VMEM((BPW,), jnp.float32),
        pltpu.VMEM((NB, CH, N_CLASSES), jnp.float32),
        pltpu.VMEM((NB, CH, N_CLASSES), jnp.float32),
        pltpu.SemaphoreType.DMA,
        [pltpu.SemaphoreType.DMA] * NB,
        [pltpu.SemaphoreType.DMA] * NB,
        [pltpu.SemaphoreType.DMA] * NB,
    ],
)
def _calibrate(logits_hbm, topics_hbm, loga_hbm, b_hbm, out_hbm,
               idx_all, loga_all, x_v, b_v, sem_l, sem_x, sem_b, sem_o):
    wid = lax.axis_index("s") * NC + lax.axis_index("c")
    base = wid * BPW

    pltpu.sync_copy(topics_hbm.at[pl.ds(base, BPW)], idx_all)
    cp_l = pltpu.async_copy(loga_hbm.at[idx_all], loga_all, sem_l)

    outs = [None] * NB

    def issue(ch):
        k = ch % NB
        if outs[k] is not None:
            outs[k].wait()
            outs[k] = None
        n = CHS[ch]
        off = base + OFFS[ch]
        cpx = pltpu.async_copy(logits_hbm.at[pl.ds(off, n)],
                               x_v.at[k].at[pl.ds(0, n)], sem_x[k])
        cpb = pltpu.async_copy(b_hbm.at[idx_all.at[pl.ds(OFFS[ch], n)]],
                               b_v.at[k].at[pl.ds(0, n)], sem_b[k])
        return cpx, cpb

    pending = {}
    for ch in range(min(AHEAD, NCH)):
        pending[ch] = issue(ch)
    cp_l.wait()
    for ch in range(NCH):
        if ch + AHEAD < NCH:
            pending[ch + AHEAD] = issue(ch + AHEAD)
        k = ch % NB
        cpx, cpb = pending.pop(ch)
        cpx.wait()
        cpb.wait()

        def pair_body(i, carry):
            r0 = i * 2
            g = (r0 // L) * L
            sv = jnp.exp(loga_all[pl.ds(OFFS[ch] + g, L)])
            for j in range(2):
                r = r0 + j
                iv = jnp.broadcast_to(r - g, (L,))
                s = lax.gather(
                    sv, iv[:, None],
                    dimension_numbers=lax.GatherDimensionNumbers(
                        offset_dims=(), collapsed_slice_dims=(0,),
                        start_index_map=(0,)),
                    slice_sizes=(1,),
                    mode=lax.GatherScatterMode.PROMISE_IN_BOUNDS)
                for c in range(CREG):
                    sl = (k, r, pl.ds(c * L, L))
                    plsc.addupdate(b_v.at[sl], x_v[sl] * s)
            return carry

        lax.fori_loop(0, CHS[ch] // 2, pair_body, 0)
        outs[k] = pltpu.async_copy(b_v.at[k].at[pl.ds(0, CHS[ch])],
                                   out_hbm.at[pl.ds(base + OFFS[ch], CHS[ch])],
                                   sem_o[k])
    for cp in outs:
        if cp is not None:
            cp.wait()


def kernel(logits, topics, loga, b):
    if topics.dtype != jnp.int32:
        topics = topics.astype(jnp.int32)
    return _calibrate(logits, topics, loga, b)


# speedup vs baseline: 1.0269x; 1.0056x over previous
import functools

import jax
import jax.numpy as jnp
from jax import lax
from jax.experimental import pallas as pl
from jax.experimental.pallas import tpu as pltpu
from jax.experimental.pallas import tpu_sc as plsc

N_TOPICS = 100000
N_CLASSES = 128
BATCH = 16384

NC, NS, L = 2, 16, 16
NW = NC * NS
BPW = BATCH // NW
CH = 128
CHS = (32, 32, 128, 128, 128, 64)
OFFS = (0, 32, 64, 192, 320, 448)
NCH = len(CHS)
NB = 3
AHEAD = 2
CREG = N_CLASSES // L

_mesh = plsc.VectorSubcoreMesh(core_axis_name="c", subcore_axis_name="s")


@functools.partial(
    pl.kernel,
    out_type=jax.ShapeDtypeStruct((BATCH, N_CLASSES), jnp.float32),
    mesh=_mesh,
    scratch_types=[
        pltpu.VMEM((BPW,), jnp.int32),
        pltpu.VMEM((BPW,), jnp.float32),
        pltpu.VMEM((NB, CH, N_CLASSES), jnp.float32),
        pltpu.VMEM((NB, CH, N_CLASSES), jnp.float32),
        pltpu.SemaphoreType.DMA,
        pltpu.SemaphoreType.DMA,
        [pltpu.SemaphoreType.DMA] * NB,
        [pltpu.SemaphoreType.DMA] * NB,
        [pltpu.SemaphoreType.DMA] * NB,
    ],
)
def _calibrate(logits_hbm, topics_hbm, loga_hbm, b_hbm, out_hbm,
               idx_all, loga_all, x_v, b_v, sem_t, sem_l, sem_x, sem_b, sem_o):
    wid = lax.axis_index("s") * NC + lax.axis_index("c")
    base = wid * BPW

    cp_t = pltpu.async_copy(topics_hbm.at[pl.ds(base, BPW)], idx_all, sem_t)

    outs = [None] * NB

    def issue_x(ch):
        k = ch % NB
        if outs[k] is not None:
            outs[k].wait()
            outs[k] = None
        n = CHS[ch]
        return pltpu.async_copy(logits_hbm.at[pl.ds(base + OFFS[ch], n)],
                                x_v.at[k].at[pl.ds(0, n)], sem_x[k])

    def issue_b(ch):
        k = ch % NB
        n = CHS[ch]
        return pltpu.async_copy(b_hbm.at[idx_all.at[pl.ds(OFFS[ch], n)]],
                                b_v.at[k].at[pl.ds(0, n)], sem_b[k])

    def issue(ch):
        return issue_x(ch), issue_b(ch)

    early_x = [issue_x(ch) for ch in range(min(AHEAD, NCH))]
    cp_t.wait()
    cp_l = pltpu.async_copy(loga_hbm.at[idx_all], loga_all, sem_l)
    pending = {}
    for ch in range(min(AHEAD, NCH)):
        pending[ch] = (early_x[ch], issue_b(ch))
    cp_l.wait()
    for ch in range(NCH):
        if ch + AHEAD < NCH:
            pending[ch + AHEAD] = issue(ch + AHEAD)
        k = ch % NB
        cpx, cpb = pending.pop(ch)
        cpx.wait()
        cpb.wait()

        def pair_body(i, carry):
            r0 = i * 2
            g = (r0 // L) * L
            sv = jnp.exp(loga_all[pl.ds(OFFS[ch] + g, L)])
            for j in range(2):
                r = r0 + j
                iv = jnp.broadcast_to(r - g, (L,))
                s = lax.gather(
                    sv, iv[:, None],
                    dimension_numbers=lax.GatherDimensionNumbers(
                        offset_dims=(), collapsed_slice_dims=(0,),
                        start_index_map=(0,)),
                    slice_sizes=(1,),
                    mode=lax.GatherScatterMode.PROMISE_IN_BOUNDS)
                for c in range(CREG):
                    sl = (k, r, pl.ds(c * L, L))
                    plsc.addupdate(b_v.at[sl], x_v[sl] * s)
            return carry

        lax.fori_loop(0, CHS[ch] // 2, pair_body, 0)
        outs[k] = pltpu.async_copy(b_v.at[k].at[pl.ds(0, CHS[ch])],
                                   out_hbm.at[pl.ds(base + OFFS[ch], CHS[ch])],
                                   sem_o[k])
    for cp in outs:
        if cp is not None:
            cp.wait()


def kernel(logits, topics, loga, b):
    if topics.dtype != jnp.int32:
        topics = topics.astype(jnp.int32)
    return _calibrate(logits, topics, loga, b)


# speedup vs baseline: 1.0424x; 1.0151x over previous
import functools

import jax
import jax.numpy as jnp
from jax import lax
from jax.experimental import pallas as pl
from jax.experimental.pallas import tpu as pltpu
from jax.experimental.pallas import tpu_sc as plsc

N_TOPICS = 100000
N_CLASSES = 128
BATCH = 16384

NC, NS, L = 2, 16, 16
NW = NC * NS
BPW = BATCH // NW
CH = 128
CHS = (64, 64, 128, 128, 64, 64)
OFFS = (0, 64, 128, 256, 384, 448)
NCH = len(CHS)
NB = 3
AHEAD = 2
CREG = N_CLASSES // L

_mesh = plsc.VectorSubcoreMesh(core_axis_name="c", subcore_axis_name="s")


@functools.partial(
    pl.kernel,
    out_type=jax.ShapeDtypeStruct((BATCH, N_CLASSES), jnp.float32),
    mesh=_mesh,
    scratch_types=[
        pltpu.VMEM((BPW,), jnp.int32),
        pltpu.VMEM((BPW,), jnp.float32),
        pltpu.VMEM((NB, CH, N_CLASSES), jnp.float32),
        pltpu.VMEM((NB, CH, N_CLASSES), jnp.float32),
        pltpu.SemaphoreType.DMA,
        pltpu.SemaphoreType.DMA,
        [pltpu.SemaphoreType.DMA] * NB,
        [pltpu.SemaphoreType.DMA] * NB,
        [pltpu.SemaphoreType.DMA] * NB,
    ],
)
def _calibrate(logits_hbm, topics_hbm, loga_hbm, b_hbm, out_hbm,
               idx_all, loga_all, x_v, b_v, sem_t, sem_l, sem_x, sem_b, sem_o):
    wid = lax.axis_index("s") * NC + lax.axis_index("c")
    base = wid * BPW

    cp_t = pltpu.async_copy(topics_hbm.at[pl.ds(base, BPW)], idx_all, sem_t)

    outs = [None] * NB

    def issue_x(ch):
        k = ch % NB
        if outs[k] is not None:
            outs[k].wait()
            outs[k] = None
        n = CHS[ch]
        return pltpu.async_copy(logits_hbm.at[pl.ds(base + OFFS[ch], n)],
                                x_v.at[k].at[pl.ds(0, n)], sem_x[k])

    def issue_b(ch):
        k = ch % NB
        n = CHS[ch]
        return pltpu.async_copy(b_hbm.at[idx_all.at[pl.ds(OFFS[ch], n)]],
                                b_v.at[k].at[pl.ds(0, n)], sem_b[k])

    def issue(ch):
        return issue_x(ch), issue_b(ch)

    early_x = [issue_x(ch) for ch in range(min(AHEAD, NCH))]
    cp_t.wait()
    cp_l = pltpu.async_copy(loga_hbm.at[idx_all], loga_all, sem_l)
    pending = {}
    for ch in range(min(AHEAD, NCH)):
        pending[ch] = (early_x[ch], issue_b(ch))
    cp_l.wait()
    for ch in range(NCH):
        if ch + AHEAD < NCH:
            pending[ch + AHEAD] = issue(ch + AHEAD)
        k = ch % NB
        cpx, cpb = pending.pop(ch)
        cpx.wait()
        cpb.wait()

        def pair_body(i, carry):
            r0 = i * 2
            g = (r0 // L) * L
            sv = jnp.exp(loga_all[pl.ds(OFFS[ch] + g, L)])
            for j in range(2):
                r = r0 + j
                iv = jnp.broadcast_to(r - g, (L,))
                s = lax.gather(
                    sv, iv[:, None],
                    dimension_numbers=lax.GatherDimensionNumbers(
                        offset_dims=(), collapsed_slice_dims=(0,),
                        start_index_map=(0,)),
                    slice_sizes=(1,),
                    mode=lax.GatherScatterMode.PROMISE_IN_BOUNDS)
                for c in range(CREG):
                    sl = (k, r, pl.ds(c * L, L))
                    plsc.addupdate(b_v.at[sl], x_v[sl] * s)
            return carry

        lax.fori_loop(0, CHS[ch] // 2, pair_body, 0)
        outs[k] = pltpu.async_copy(b_v.at[k].at[pl.ds(0, CHS[ch])],
                                   out_hbm.at[pl.ds(base + OFFS[ch], CHS[ch])],
                                   sem_o[k])
    for cp in outs:
        if cp is not None:
            cp.wait()


def kernel(logits, topics, loga, b):
    if topics.dtype != jnp.int32:
        topics = topics.astype(jnp.int32)
    return _calibrate(logits, topics, loga, b)
